# baseline (speedup 1.0000x reference)
import functools

import jax
import jax.numpy as jnp
from jax import lax
from jax.experimental import pallas as pl
from jax.experimental.pallas import tpu as pltpu
from jax.experimental.pallas import tpu_sc as plsc

B = 1024
D = 16
N_USERS = 1000000
N_ITEMS = 100000
NC = 2
NS = 16
NW = NC * NS
EPW = B * D // NW
CH = 128

IB = 2048


@functools.partial(
    pl.kernel,
    out_type=jax.ShapeDtypeStruct((D * B,), jnp.float32),
    mesh=plsc.VectorSubcoreMesh(core_axis_name="c", subcore_axis_name="s"),
    compiler_params=pltpu.CompilerParams(use_tc_tiling_on_sc=False),
    scratch_types=[
        pltpu.VMEM((EPW,), jnp.int32),
        pltpu.VMEM((EPW,), jnp.float32),
        pltpu.SemaphoreType.DMA,
    ],
)
def _sc_gather(flat_t, idx_hbm, out_hbm, idx_v, vals_v, sem):
    wid = lax.axis_index("s") * NC + lax.axis_index("c")
    base = wid * EPW
    pltpu.sync_copy(idx_hbm.at[pl.ds(base, EPW)], idx_v)
    copies = [
        pltpu.async_copy(
            flat_t.at[idx_v.at[pl.ds(c * CH, CH)]],
            vals_v.at[pl.ds(c * CH, CH)],
            sem,
        )
        for c in range(EPW // CH)
    ]
    for cp in copies:
        cp.wait()
    pltpu.sync_copy(vals_v, out_hbm.at[pl.ds(base, EPW)])


def _mm_body(it_ref, ut_ref, out_ref):
    out_ref[...] = lax.dot_general(
        it_ref[...], ut_ref[...],
        dimension_numbers=(((0,), (0,)), ((), ())),
        preferred_element_type=jnp.float32,
    )


def _tc_matmul(item_t, ut):
    return pl.pallas_call(
        _mm_body,
        grid=(pl.cdiv(N_ITEMS, IB),),
        in_specs=[
            pl.BlockSpec((D, IB), lambda i: (0, i)),
            pl.BlockSpec((D, B), lambda i: (0, 0)),
        ],
        out_specs=pl.BlockSpec((IB, B), lambda i: (i, 0)),
        out_shape=jax.ShapeDtypeStruct((N_ITEMS, B), jnp.float32),
    )(item_t, ut)


@jax.jit
def kernel(users, user_table, item_table):
    r = users.reshape(-1).astype(jnp.int32)
    idx = (jnp.arange(D, dtype=jnp.int32) * N_USERS)[:, None] + r[None, :]
    flat_t = user_table.T.reshape(-1)
    ut = _sc_gather(flat_t, idx.reshape(-1)).reshape(D, B)
    scores_t = _tc_matmul(item_table.T, ut)
    return scores_t.T


# speedup vs baseline: 9.7442x; 9.7442x over previous
import jax
import jax.numpy as jnp
from jax import lax
from jax.experimental import pallas as pl
from jax.experimental.pallas import tpu as pltpu

B = 1024
D = 16
N_USERS = 1000000
N_ITEMS = 100000
LANES = 128
IB = 2048
GRID = pl.cdiv(N_ITEMS, IB)


def _body(users_smem, lane_ref, table_t_hbm, it_ref, out_ref,
          blk_vmem, u_vmem, sem):
    i = pl.program_id(0)

    @pl.when(i == 0)
    def _gather():
        def copy_j(j):
            r = users_smem[j]
            q = lax.div(r, LANES) * LANES
            return pltpu.make_async_copy(
                table_t_hbm.at[:, pl.ds(q, LANES)],
                blk_vmem.at[j],
                sem,
            )

        lax.fori_loop(0, B, lambda j, _: (copy_j(j).start(), _)[1], None)
        lax.fori_loop(0, B, lambda j, _: (copy_j(j).wait(), _)[1], None)
        onehot = (lax.broadcasted_iota(jnp.int32, (B, 1, LANES), 2)
                  == lane_ref[...].reshape(B, 1, 1)).astype(jnp.float32)
        u_vmem[...] = jnp.sum(blk_vmem[...] * onehot, axis=2)

    out_ref[...] = lax.dot_general(
        it_ref[...], u_vmem[...],
        dimension_numbers=(((0,), (1,)), ((), ())),
        preferred_element_type=jnp.float32,
    )


def _fused(users_i32, lane_i32, item_t, table_t):
    return pl.pallas_call(
        _body,
        grid=(GRID,),
        in_specs=[
            pl.BlockSpec(memory_space=pltpu.MemorySpace.SMEM),
            pl.BlockSpec((B,), lambda i: (0,)),
            pl.BlockSpec(memory_space=pltpu.MemorySpace.HBM),
            pl.BlockSpec((D, IB), lambda i: (0, i)),
        ],
        out_specs=pl.BlockSpec((IB, B), lambda i: (i, 0)),
        out_shape=jax.ShapeDtypeStruct((N_ITEMS, B), jnp.float32),
        scratch_shapes=[
            pltpu.VMEM((B, D, LANES), jnp.float32),
            pltpu.VMEM((B, D), jnp.float32),
            pltpu.SemaphoreType.DMA,
        ],
    )(users_i32, lane_i32, table_t, item_t)


@jax.jit
def kernel(users, user_table, item_table):
    users_i32 = users.reshape(-1).astype(jnp.int32)
    lane_i32 = users_i32 % LANES
    scores_t = _fused(users_i32, lane_i32, item_table.T, user_table.T)
    return scores_t.T
